# baseline (speedup 1.0000x reference)
import functools

import jax
import jax.numpy as jnp
from jax import lax
from jax.experimental import pallas as pl
from jax.experimental.pallas import tpu as pltpu
from jax.experimental.pallas import tpu_sc as plsc

_SELU_ALPHA = 1.6732632423543772848170429916717
_SELU_SCALE = 1.0507009873554804934193349852946

_TG = 128


def _selu(v):
    return _SELU_SCALE * jnp.where(v > 0, v, _SELU_ALPHA * (jnp.exp(v) - 1.0))


def _gate_body(x_ref, gwi_ref, gbi_ref, gwo_ref, gbo_ref, wt_ref,
               *, n_experts):
    x = x_ref[...]
    h = _selu(
        jnp.dot(x, gwi_ref[...], preferred_element_type=jnp.float32)
        + gbi_ref[...])
    logits = (jnp.dot(h, gwo_ref[...], preferred_element_type=jnp.float32)
              + gbo_ref[...])
    ta, hdim = logits.shape
    iota = jax.lax.broadcasted_iota(jnp.int32, (ta, hdim), 1)
    m1 = jnp.max(logits, axis=1, keepdims=True)
    i1 = jnp.min(jnp.where(logits == m1, iota, hdim), axis=1, keepdims=True)
    neg = jnp.finfo(jnp.float32).min
    masked = jnp.where(iota == i1, neg, logits)
    m2 = jnp.max(masked, axis=1, keepdims=True)
    i2 = jnp.min(jnp.where(masked == m2, iota, hdim), axis=1, keepdims=True)
    s = m1 + m2
    p1 = m1 / s
    p2 = m2 / s
    eiota = jax.lax.broadcasted_iota(jnp.int32, (ta, n_experts), 1)
    hit1 = i1 == eiota
    hit2 = i2 == eiota
    w = (p1 * hit1.astype(jnp.float32) + p2 * hit2.astype(jnp.float32))
    wt_ref[...] = w.T


def _make_router(n, n_experts, npad, n_cores):
    mesh = plsc.VectorSubcoreMesh(core_axis_name="c", subcore_axis_name="s")

    @functools.partial(
        pl.kernel, mesh=mesh,
        out_type=(
            jax.ShapeDtypeStruct((n_experts, npad), jnp.int32),
            jax.ShapeDtypeStruct((n_experts, npad), jnp.float32),
            jax.ShapeDtypeStruct((n_experts, 16), jnp.int32),
        ),
        scratch_types=[
            pltpu.VMEM((n,), jnp.float32),
            pltpu.VMEM((npad,), jnp.int32),
            pltpu.VMEM((npad,), jnp.float32),
            pltpu.VMEM((16,), jnp.int32),
        ],
    )
    def router(wt_hbm, toks_hbm, wl_hbm, cnt_hbm, wrow, tbuf, wbuf, cbuf):
        wid = lax.axis_index("s") * n_cores + lax.axis_index("c")

        @pl.when(wid < n_experts)
        def _worker():
            pltpu.sync_copy(wt_hbm.at[wid], wrow)
            zi16 = jnp.zeros((16,), jnp.int32)
            zf16 = jnp.zeros((16,), jnp.float32)

            def chunk(i, cur):
                v = wrow[pl.ds(i * 16, 16)]
                for l in range(16):
                    w_l = v[l]
                    hit = w_l != 0.0

                    @pl.when(hit)
                    def _(cur=cur, w_l=w_l, i=i, l=l):
                        tbuf[pl.ds(cur, 16)] = zi16 + (i * 16 + l)
                        wbuf[pl.ds(cur, 16)] = zf16 + w_l

                    cur = cur + jnp.where(hit, 1, 0)
                return cur

            cnt = lax.fori_loop(0, n // 16, chunk, 0)
            for j in range(_TG // 16):
                tbuf[pl.ds(cnt + j * 16, 16)] = zi16
                wbuf[pl.ds(cnt + j * 16, 16)] = zf16
            cbuf[...] = zi16 + cnt
            pltpu.sync_copy(tbuf, toks_hbm.at[wid])
            pltpu.sync_copy(wbuf, wl_hbm.at[wid])
            pltpu.sync_copy(cbuf, cnt_hbm.at[wid])

    return router


def _expert_body(counts_ref, toks_ref, x_ref, wl_ref, wi_ref, bi_ref, wo_ref,
                 bo_ref, out_ref, xb_ref, xg_ref, yacc_ref,
                 *, nf, ntiles_max, n):
    e = pl.program_id(0)
    f = pl.program_id(1)

    @pl.when((e == 0) & (f == 0))
    def _init():
        out_ref[...] = jnp.zeros_like(out_ref)
        xb_ref[...] = x_ref[...].astype(jnp.bfloat16)

    cnt = counts_ref[e, 0]
    ntiles = (cnt + _TG - 1) // _TG
    wi = wi_ref[0].astype(jnp.bfloat16)
    wo = wo_ref[0].astype(jnp.bfloat16)
    bi = bi_ref[0]
    bo = bo_ref[0]
    keep = jnp.where(f == 0, 0.0, 1.0)
    sel = (jax.lax.broadcasted_iota(jnp.int32, (_TG, _TG), 0)
           == jax.lax.broadcasted_iota(jnp.int32, (_TG, _TG), 1))
    iota_row = jax.lax.broadcasted_iota(jnp.int32, (_TG, n), 1)
    iota_col = jax.lax.broadcasted_iota(jnp.int32, (n, _TG), 0)

    def tile_body(j, carry):
        base = j * _TG
        q = e * ntiles_max + j
        trow = toks_ref[pl.ds(q, 1)].reshape(1, _TG)

        @pl.when(f == 0)
        def _gather():
            tcol = jnp.sum(jnp.where(sel, trow, 0), axis=1, keepdims=True)
            p = (tcol == iota_row).astype(jnp.bfloat16)
            xg_ref[pl.ds(base, _TG), :] = jnp.dot(
                p, xb_ref[...],
                preferred_element_type=jnp.float32).astype(jnp.bfloat16)

        xs = xg_ref[pl.ds(base, _TG), :]
        h = _selu(
            jnp.dot(xs, wi, preferred_element_type=jnp.float32) + bi)
        hw = jnp.dot(h.astype(jnp.bfloat16), wo,
                     preferred_element_type=jnp.float32)
        yacc_ref[pl.ds(base, _TG), :] = (
            yacc_ref[pl.ds(base, _TG), :] * keep + hw)

        @pl.when(f == nf - 1)
        def _scatter():
            wrow = wl_ref[pl.ds(q, 1)].reshape(1, _TG)
            wcol = jnp.sum(jnp.where(sel, wrow, 0.0), axis=1, keepdims=True)
            ys = (yacc_ref[pl.ds(base, _TG), :] + bo) * wcol
            pt = (iota_col == trow).astype(jnp.bfloat16)
            out_ref[...] += jnp.dot(pt, ys.astype(jnp.bfloat16),
                                    preferred_element_type=jnp.float32)

        return carry

    lax.fori_loop(0, ntiles, tile_body, 0)


def kernel(x, gate_wi, gate_bi, gate_wo, gate_bo, exp_wi, exp_bi, exp_wo,
           exp_bo):
    b, s, hdim = x.shape
    n = b * s
    e_num, _, fdim = exp_wi.shape
    x_flat = x.reshape(n, hdim)

    tile_a = 256 if n % 256 == 0 else n
    f_chunk = 512 if fdim % 512 == 0 else fdim
    nf = fdim // f_chunk
    npad = n + 2 * _TG

    gate = pl.pallas_call(
        functools.partial(_gate_body, n_experts=e_num),
        grid=(n // tile_a,),
        in_specs=[
            pl.BlockSpec((tile_a, hdim), lambda t: (t, 0)),
            pl.BlockSpec((hdim, fdim), lambda t: (0, 0)),
            pl.BlockSpec((1, fdim), lambda t: (0, 0)),
            pl.BlockSpec((fdim, hdim), lambda t: (0, 0)),
            pl.BlockSpec((1, hdim), lambda t: (0, 0)),
        ],
        out_specs=pl.BlockSpec((e_num, tile_a), lambda t: (0, t)),
        out_shape=jax.ShapeDtypeStruct((e_num, n), jnp.float32),
    )
    w_t = gate(x_flat, gate_wi, gate_bi.reshape(1, fdim),
               gate_wo, gate_bo.reshape(1, hdim))

    info = plsc.get_sparse_core_info()
    router = _make_router(n, e_num, npad, info.num_cores)
    toks, wlist, counts = router(w_t)

    ntiles_max = n // _TG
    nq = e_num * ntiles_max
    toks3 = toks[:, :n].reshape(nq, 1, _TG)
    wl3 = wlist[:, :n].reshape(nq, 1, _TG)

    expert = pl.pallas_call(
        functools.partial(_expert_body, nf=nf, ntiles_max=ntiles_max, n=n),
        grid=(e_num, nf),
        in_specs=[
            pl.BlockSpec(memory_space=pltpu.SMEM),
            pl.BlockSpec((nq, 1, _TG), lambda e, f: (0, 0, 0)),
            pl.BlockSpec((n, hdim), lambda e, f: (0, 0)),
            pl.BlockSpec((nq, 1, _TG), lambda e, f: (0, 0, 0)),
            pl.BlockSpec((1, hdim, f_chunk), lambda e, f: (e, 0, f)),
            pl.BlockSpec((1, 1, f_chunk), lambda e, f: (e, 0, f)),
            pl.BlockSpec((1, f_chunk, hdim), lambda e, f: (e, f, 0)),
            pl.BlockSpec((1, 1, hdim), lambda e, f: (e, 0, 0)),
        ],
        out_specs=pl.BlockSpec((n, hdim), lambda e, f: (0, 0)),
        out_shape=jax.ShapeDtypeStruct((n, hdim), jnp.float32),
        scratch_shapes=[
            pltpu.VMEM((n, hdim), jnp.bfloat16),
            pltpu.VMEM((n, hdim), jnp.bfloat16),
            pltpu.VMEM((n, hdim), jnp.float32),
        ],
    )
    out = expert(counts, toks3, x_flat, wl3, exp_wi,
                 exp_bi.reshape(e_num, 1, fdim), exp_wo,
                 exp_bo.reshape(e_num, 1, hdim))
    return out.reshape(b, s, hdim)


# speedup vs baseline: 1.1449x; 1.1449x over previous
import functools

import jax
import jax.numpy as jnp
from jax import lax
from jax.experimental import pallas as pl
from jax.experimental.pallas import tpu as pltpu
from jax.experimental.pallas import tpu_sc as plsc

_SELU_ALPHA = 1.6732632423543772848170429916717
_SELU_SCALE = 1.0507009873554804934193349852946

_TG = 128


def _selu(v):
    return _SELU_SCALE * jnp.where(v > 0, v, _SELU_ALPHA * (jnp.exp(v) - 1.0))


def _gate_body(x_ref, gwi_ref, gbi_ref, gwo_ref, gbo_ref, wt_ref,
               *, n_experts):
    x = x_ref[...]
    h = _selu(
        jnp.dot(x, gwi_ref[...], preferred_element_type=jnp.float32)
        + gbi_ref[...])
    logits = (jnp.dot(h, gwo_ref[...], preferred_element_type=jnp.float32)
              + gbo_ref[...])
    ta, hdim = logits.shape
    iota = jax.lax.broadcasted_iota(jnp.int32, (ta, hdim), 1)
    m1 = jnp.max(logits, axis=1, keepdims=True)
    i1 = jnp.min(jnp.where(logits == m1, iota, hdim), axis=1, keepdims=True)
    neg = jnp.finfo(jnp.float32).min
    masked = jnp.where(iota == i1, neg, logits)
    m2 = jnp.max(masked, axis=1, keepdims=True)
    i2 = jnp.min(jnp.where(masked == m2, iota, hdim), axis=1, keepdims=True)
    s = m1 + m2
    p1 = m1 / s
    p2 = m2 / s
    eiota = jax.lax.broadcasted_iota(jnp.int32, (ta, n_experts), 1)
    hit1 = i1 == eiota
    hit2 = i2 == eiota
    w = (p1 * hit1.astype(jnp.float32) + p2 * hit2.astype(jnp.float32))
    wt_ref[...] = w.T


def _make_router(n, n_experts, npad, n_cores):
    mesh = plsc.VectorSubcoreMesh(core_axis_name="c", subcore_axis_name="s")

    @functools.partial(
        pl.kernel, mesh=mesh,
        out_type=(
            jax.ShapeDtypeStruct((n_experts, npad), jnp.int32),
            jax.ShapeDtypeStruct((n_experts, npad), jnp.float32),
            jax.ShapeDtypeStruct((n_experts, 16), jnp.int32),
        ),
        scratch_types=[
            pltpu.VMEM((n,), jnp.float32),
            pltpu.VMEM((npad,), jnp.int32),
            pltpu.VMEM((npad,), jnp.float32),
            pltpu.VMEM((16,), jnp.int32),
        ],
    )
    def router(wt_hbm, toks_hbm, wl_hbm, cnt_hbm, wrow, tbuf, wbuf, cbuf):
        wid = lax.axis_index("s") * n_cores + lax.axis_index("c")

        @pl.when(wid < n_experts)
        def _worker():
            pltpu.sync_copy(wt_hbm.at[wid], wrow)
            zi16 = jnp.zeros((16,), jnp.int32)
            zf16 = jnp.zeros((16,), jnp.float32)

            def chunk(i, cur):
                v = wrow[pl.ds(i * 16, 16)]
                for l in range(16):
                    w_l = v[l]
                    hit = w_l != 0.0

                    @pl.when(hit)
                    def _(cur=cur, w_l=w_l, i=i, l=l):
                        tbuf[pl.ds(cur, 16)] = zi16 + (i * 16 + l)
                        wbuf[pl.ds(cur, 16)] = zf16 + w_l

                    cur = cur + jnp.where(hit, 1, 0)
                return cur

            cnt = lax.fori_loop(0, n // 16, chunk, 0)
            for j in range(_TG // 16):
                tbuf[pl.ds(cnt + j * 16, 16)] = zi16
                wbuf[pl.ds(cnt + j * 16, 16)] = zf16
            cbuf[...] = zi16 + cnt
            pltpu.sync_copy(tbuf, toks_hbm.at[wid])
            pltpu.sync_copy(wbuf, wl_hbm.at[wid])
            pltpu.sync_copy(cbuf, cnt_hbm.at[wid])

    return router


def _expert_body(counts_ref, toks_ref, x_ref, wl_ref, wi_ref, bi_ref, wo_ref,
                 bo_ref, out_ref, xb_ref, xg_ref, yacc_ref,
                 *, nf, ntiles_max, n):
    e = pl.program_id(0)
    f = pl.program_id(1)

    @pl.when((e == 0) & (f == 0))
    def _init():
        out_ref[...] = jnp.zeros_like(out_ref)
        xb_ref[...] = x_ref[...].astype(jnp.bfloat16)

    cnt = counts_ref[e, 0]
    ntiles = (cnt + _TG - 1) // _TG
    wi = wi_ref[0].astype(jnp.bfloat16)
    wo = wo_ref[0].astype(jnp.bfloat16)
    bi = bi_ref[0]
    bo = bo_ref[0]
    keep = jnp.where(f == 0, 0.0, 1.0)
    sel = (jax.lax.broadcasted_iota(jnp.int32, (_TG, _TG), 0)
           == jax.lax.broadcasted_iota(jnp.int32, (_TG, _TG), 1))
    iota_row = jax.lax.broadcasted_iota(jnp.int32, (_TG, n), 1)
    iota_col = jax.lax.broadcasted_iota(jnp.int32, (n, _TG), 0)

    def tile_body(j, carry):
        base = j * _TG
        q = e * ntiles_max + j
        trow = toks_ref[pl.ds(q, 1)].reshape(1, _TG)

        @pl.when(f == 0)
        def _gather():
            tcol = jnp.sum(jnp.where(sel, trow, 0), axis=1, keepdims=True)
            p = (tcol == iota_row).astype(jnp.bfloat16)
            xg_ref[pl.ds(base, _TG), :] = jnp.dot(
                p, xb_ref[...],
                preferred_element_type=jnp.float32).astype(jnp.bfloat16)

        xs = xg_ref[pl.ds(base, _TG), :]
        h = _selu(
            jnp.dot(xs, wi, preferred_element_type=jnp.float32) + bi)
        hw = jnp.dot(h.astype(jnp.bfloat16), wo,
                     preferred_element_type=jnp.float32)
        yacc_ref[pl.ds(base, _TG), :] = (
            yacc_ref[pl.ds(base, _TG), :] * keep + hw)

        @pl.when(f == nf - 1)
        def _scatter():
            wrow = wl_ref[pl.ds(q, 1)].reshape(1, _TG)
            wcol = jnp.sum(jnp.where(sel, wrow, 0.0), axis=1, keepdims=True)
            ys = (yacc_ref[pl.ds(base, _TG), :] + bo) * wcol
            pt = (iota_col == trow).astype(jnp.bfloat16)
            out_ref[...] += jnp.dot(pt, ys.astype(jnp.bfloat16),
                                    preferred_element_type=jnp.float32)

        return carry

    lax.fori_loop(0, ntiles, tile_body, 0)


def kernel(x, gate_wi, gate_bi, gate_wo, gate_bo, exp_wi, exp_bi, exp_wo,
           exp_bo):
    b, s, hdim = x.shape
    n = b * s
    e_num, _, fdim = exp_wi.shape
    x_flat = x.reshape(n, hdim)

    tile_a = 512 if n % 512 == 0 else n
    f_chunk = 1024 if fdim % 1024 == 0 else fdim
    nf = fdim // f_chunk
    npad = n + 2 * _TG

    gate = pl.pallas_call(
        functools.partial(_gate_body, n_experts=e_num),
        grid=(n // tile_a,),
        in_specs=[
            pl.BlockSpec((tile_a, hdim), lambda t: (t, 0)),
            pl.BlockSpec((hdim, fdim), lambda t: (0, 0)),
            pl.BlockSpec((1, fdim), lambda t: (0, 0)),
            pl.BlockSpec((fdim, hdim), lambda t: (0, 0)),
            pl.BlockSpec((1, hdim), lambda t: (0, 0)),
        ],
        out_specs=pl.BlockSpec((e_num, tile_a), lambda t: (0, t)),
        out_shape=jax.ShapeDtypeStruct((e_num, n), jnp.float32),
    )
    w_t = gate(x_flat, gate_wi, gate_bi.reshape(1, fdim),
               gate_wo, gate_bo.reshape(1, hdim))

    info = plsc.get_sparse_core_info()
    router = _make_router(n, e_num, npad, info.num_cores)
    toks, wlist, counts = router(w_t)

    ntiles_max = n // _TG
    nq = e_num * ntiles_max
    toks3 = toks[:, :n].reshape(nq, 1, _TG)
    wl3 = wlist[:, :n].reshape(nq, 1, _TG)

    expert = pl.pallas_call(
        functools.partial(_expert_body, nf=nf, ntiles_max=ntiles_max, n=n),
        grid=(e_num, nf),
        in_specs=[
            pl.BlockSpec(memory_space=pltpu.SMEM),
            pl.BlockSpec((nq, 1, _TG), lambda e, f: (0, 0, 0)),
            pl.BlockSpec((n, hdim), lambda e, f: (0, 0)),
            pl.BlockSpec((nq, 1, _TG), lambda e, f: (0, 0, 0)),
            pl.BlockSpec((1, hdim, f_chunk), lambda e, f: (e, 0, f)),
            pl.BlockSpec((1, 1, f_chunk), lambda e, f: (e, 0, f)),
            pl.BlockSpec((1, f_chunk, hdim), lambda e, f: (e, f, 0)),
            pl.BlockSpec((1, 1, hdim), lambda e, f: (e, 0, 0)),
        ],
        out_specs=pl.BlockSpec((n, hdim), lambda e, f: (0, 0)),
        out_shape=jax.ShapeDtypeStruct((n, hdim), jnp.float32),
        scratch_shapes=[
            pltpu.VMEM((n, hdim), jnp.bfloat16),
            pltpu.VMEM((n, hdim), jnp.bfloat16),
            pltpu.VMEM((n, hdim), jnp.float32),
        ],
    )
    out = expert(counts, toks3, x_flat, wl3, exp_wi,
                 exp_bi.reshape(e_num, 1, fdim), exp_wo,
                 exp_bo.reshape(e_num, 1, hdim))
    return out.reshape(b, s, hdim)


# speedup vs baseline: 1.1471x; 1.0019x over previous
import functools

import jax
import jax.numpy as jnp
from jax import lax
from jax.experimental import pallas as pl
from jax.experimental.pallas import tpu as pltpu
from jax.experimental.pallas import tpu_sc as plsc

_SELU_ALPHA = 1.6732632423543772848170429916717
_SELU_SCALE = 1.0507009873554804934193349852946

_TG = 128


def _selu(v):
    return _SELU_SCALE * jnp.where(v > 0, v, _SELU_ALPHA * (jnp.exp(v) - 1.0))


def _gate_body(x_ref, gwi_ref, gbi_ref, gwo_ref, gbo_ref, wt_ref,
               *, n_experts):
    x = x_ref[...]
    h = _selu(
        jnp.dot(x, gwi_ref[...], preferred_element_type=jnp.float32)
        + gbi_ref[...])
    logits = (jnp.dot(h, gwo_ref[...], preferred_element_type=jnp.float32)
              + gbo_ref[...])
    ta, hdim = logits.shape
    iota = jax.lax.broadcasted_iota(jnp.int32, (ta, hdim), 1)
    m1 = jnp.max(logits, axis=1, keepdims=True)
    i1 = jnp.min(jnp.where(logits == m1, iota, hdim), axis=1, keepdims=True)
    neg = jnp.finfo(jnp.float32).min
    masked = jnp.where(iota == i1, neg, logits)
    m2 = jnp.max(masked, axis=1, keepdims=True)
    i2 = jnp.min(jnp.where(masked == m2, iota, hdim), axis=1, keepdims=True)
    s = m1 + m2
    p1 = m1 / s
    p2 = m2 / s
    eiota = jax.lax.broadcasted_iota(jnp.int32, (ta, n_experts), 1)
    hit1 = i1 == eiota
    hit2 = i2 == eiota
    w = (p1 * hit1.astype(jnp.float32) + p2 * hit2.astype(jnp.float32))
    wt_ref[...] = w.T


def _make_router(n, n_experts, npad, n_cores):
    mesh = plsc.VectorSubcoreMesh(core_axis_name="c", subcore_axis_name="s")

    @functools.partial(
        pl.kernel, mesh=mesh,
        out_type=(
            jax.ShapeDtypeStruct((n_experts, npad), jnp.int32),
            jax.ShapeDtypeStruct((n_experts, npad), jnp.float32),
            jax.ShapeDtypeStruct((n_experts, 16), jnp.int32),
        ),
        scratch_types=[
            pltpu.VMEM((n,), jnp.float32),
            pltpu.VMEM((npad,), jnp.int32),
            pltpu.VMEM((npad,), jnp.float32),
            pltpu.VMEM((16,), jnp.int32),
        ],
    )
    def router(wt_hbm, toks_hbm, wl_hbm, cnt_hbm, wrow, tbuf, wbuf, cbuf):
        wid = lax.axis_index("s") * n_cores + lax.axis_index("c")

        @pl.when(wid < n_experts)
        def _worker():
            pltpu.sync_copy(wt_hbm.at[wid], wrow)
            zi16 = jnp.zeros((16,), jnp.int32)
            zf16 = jnp.zeros((16,), jnp.float32)

            def chunk(i, cur):
                v = wrow[pl.ds(i * 16, 16)]
                for l in range(16):
                    w_l = v[l]
                    hit = w_l != 0.0

                    @pl.when(hit)
                    def _(cur=cur, w_l=w_l, i=i, l=l):
                        tbuf[pl.ds(cur, 16)] = zi16 + (i * 16 + l)
                        wbuf[pl.ds(cur, 16)] = zf16 + w_l

                    cur = cur + jnp.where(hit, 1, 0)
                return cur

            cnt = lax.fori_loop(0, n // 16, chunk, 0)
            for j in range(_TG // 16):
                tbuf[pl.ds(cnt + j * 16, 16)] = zi16
                wbuf[pl.ds(cnt + j * 16, 16)] = zf16
            cbuf[...] = zi16 + cnt
            pltpu.sync_copy(tbuf, toks_hbm.at[wid])
            pltpu.sync_copy(wbuf, wl_hbm.at[wid])
            pltpu.sync_copy(cbuf, cnt_hbm.at[wid])

    return router


def _expert_body(counts_ref, toks_ref, x_ref, wl_ref, wi_ref, bi_ref, wo_ref,
                 bo_ref, out_ref, xb_ref, xg_ref, yacc_ref,
                 *, nf, ntiles_max, n):
    e = pl.program_id(0)
    f = pl.program_id(1)

    @pl.when((e == 0) & (f == 0))
    def _init():
        out_ref[...] = jnp.zeros_like(out_ref)
        xb_ref[...] = x_ref[...].astype(jnp.bfloat16)

    cnt = counts_ref[e, 0]
    ntiles = (cnt + _TG - 1) // _TG
    wi = wi_ref[0].astype(jnp.bfloat16)
    wo = wo_ref[0].astype(jnp.bfloat16)
    bi = bi_ref[0]
    bo = bo_ref[0]
    keep = jnp.where(f == 0, 0.0, 1.0)
    sel = (jax.lax.broadcasted_iota(jnp.int32, (_TG, _TG), 0)
           == jax.lax.broadcasted_iota(jnp.int32, (_TG, _TG), 1))
    iota_row = jax.lax.broadcasted_iota(jnp.int32, (_TG, n), 1)
    iota_col = jax.lax.broadcasted_iota(jnp.int32, (n, _TG), 0)

    def tile_body(j, carry):
        base = j * _TG
        q = e * ntiles_max + j
        trow = toks_ref[pl.ds(q, 1)].reshape(1, _TG)

        @pl.when(f == 0)
        def _gather():
            tcol = jnp.sum(jnp.where(sel, trow, 0), axis=1, keepdims=True)
            p = (tcol == iota_row).astype(jnp.bfloat16)
            xg_ref[pl.ds(base, _TG), :] = jnp.dot(
                p, xb_ref[...],
                preferred_element_type=jnp.float32).astype(jnp.bfloat16)

        xs = xg_ref[pl.ds(base, _TG), :]
        h = _selu(
            jnp.dot(xs, wi, preferred_element_type=jnp.float32) + bi)
        hw = jnp.dot(h.astype(jnp.bfloat16), wo,
                     preferred_element_type=jnp.float32)
        yacc_ref[pl.ds(base, _TG), :] = (
            yacc_ref[pl.ds(base, _TG), :] * keep + hw)

        @pl.when(f == nf - 1)
        def _scatter():
            wrow = wl_ref[pl.ds(q, 1)].reshape(1, _TG)
            wcol = jnp.sum(jnp.where(sel, wrow, 0.0), axis=1, keepdims=True)
            ys = (yacc_ref[pl.ds(base, _TG), :] + bo) * wcol
            pt = (iota_col == trow).astype(jnp.bfloat16)
            out_ref[...] += jnp.dot(pt, ys.astype(jnp.bfloat16),
                                    preferred_element_type=jnp.float32)

        return carry

    lax.fori_loop(0, ntiles, tile_body, 0)


def kernel(x, gate_wi, gate_bi, gate_wo, gate_bo, exp_wi, exp_bi, exp_wo,
           exp_bo):
    b, s, hdim = x.shape
    n = b * s
    e_num, _, fdim = exp_wi.shape
    x_flat = x.reshape(n, hdim)

    tile_a = 1024 if n % 1024 == 0 else n
    f_chunk = 1024 if fdim % 1024 == 0 else fdim
    nf = fdim // f_chunk
    npad = n + 2 * _TG

    gate = pl.pallas_call(
        functools.partial(_gate_body, n_experts=e_num),
        grid=(n // tile_a,),
        in_specs=[
            pl.BlockSpec((tile_a, hdim), lambda t: (t, 0)),
            pl.BlockSpec((hdim, fdim), lambda t: (0, 0)),
            pl.BlockSpec((1, fdim), lambda t: (0, 0)),
            pl.BlockSpec((fdim, hdim), lambda t: (0, 0)),
            pl.BlockSpec((1, hdim), lambda t: (0, 0)),
        ],
        out_specs=pl.BlockSpec((e_num, tile_a), lambda t: (0, t)),
        out_shape=jax.ShapeDtypeStruct((e_num, n), jnp.float32),
    )
    w_t = gate(x_flat, gate_wi, gate_bi.reshape(1, fdim),
               gate_wo, gate_bo.reshape(1, hdim))

    info = plsc.get_sparse_core_info()
    router = _make_router(n, e_num, npad, info.num_cores)
    toks, wlist, counts = router(w_t)

    ntiles_max = n // _TG
    nq = e_num * ntiles_max
    toks3 = toks[:, :n].reshape(nq, 1, _TG)
    wl3 = wlist[:, :n].reshape(nq, 1, _TG)

    expert = pl.pallas_call(
        functools.partial(_expert_body, nf=nf, ntiles_max=ntiles_max, n=n),
        grid=(e_num, nf),
        in_specs=[
            pl.BlockSpec(memory_space=pltpu.SMEM),
            pl.BlockSpec((nq, 1, _TG), lambda e, f: (0, 0, 0)),
            pl.BlockSpec((n, hdim), lambda e, f: (0, 0)),
            pl.BlockSpec((nq, 1, _TG), lambda e, f: (0, 0, 0)),
            pl.BlockSpec((1, hdim, f_chunk), lambda e, f: (e, 0, f)),
            pl.BlockSpec((1, 1, f_chunk), lambda e, f: (e, 0, f)),
            pl.BlockSpec((1, f_chunk, hdim), lambda e, f: (e, f, 0)),
            pl.BlockSpec((1, 1, hdim), lambda e, f: (e, 0, 0)),
        ],
        out_specs=pl.BlockSpec((n, hdim), lambda e, f: (0, 0)),
        out_shape=jax.ShapeDtypeStruct((n, hdim), jnp.float32),
        scratch_shapes=[
            pltpu.VMEM((n, hdim), jnp.bfloat16),
            pltpu.VMEM((n, hdim), jnp.bfloat16),
            pltpu.VMEM((n, hdim), jnp.float32),
        ],
    )
    out = expert(counts, toks3, x_flat, wl3, exp_wi,
                 exp_bi.reshape(e_num, 1, fdim), exp_wo,
                 exp_bo.reshape(e_num, 1, hdim))
    return out.reshape(b, s, hdim)
